# baseline (speedup 1.0000x reference)
import functools

import jax
import jax.numpy as jnp
from jax import lax
from jax.experimental import pallas as pl
from jax.experimental.pallas import tpu as pltpu
from jax.experimental.pallas import tpu_sc as plsc

BSZ, SEQ, IN_DIM, PROJ_DIM, Q = 4, 256, 768, 128, 16384
ROWS = BSZ * SEQ
TBL_ROWS = ROWS + 128
ZERO_ROW = ROWS

NC, NS = 2, 16
NW = NC * NS
QPW = Q // NW
QPI = 64
IDX_N = 2 * QPI
ISSUES = QPW // QPI
TOT = 2 * ISSUES
NBUF = 4
LOOKAHEAD = 2

_GDN = lax.GatherDimensionNumbers(
    offset_dims=(), collapsed_slice_dims=(0,), start_index_map=(0,)
)


def _vgather(x, idx):
    return lax.gather(
        x, idx[:, None], _GDN, slice_sizes=(1,),
        mode=lax.GatherScatterMode.PROMISE_IN_BOUNDS,
    )


def _proj_body(x_ref, w_ref, b_ref, o_ref):
    i = pl.program_id(0)

    @pl.when(i < ROWS // 128)
    def _():
        o_ref[...] = (
            jnp.dot(x_ref[...], w_ref[...], preferred_element_type=jnp.float32)
            + b_ref[...]
        )

    @pl.when(i >= ROWS // 128)
    def _():
        o_ref[...] = jnp.zeros_like(o_ref)


def _project(x2d, W, b2d):
    nblk = TBL_ROWS // 128
    return pl.pallas_call(
        _proj_body,
        grid=(nblk,),
        in_specs=[
            pl.BlockSpec((128, IN_DIM), lambda i: (jnp.minimum(i, ROWS // 128 - 1), 0)),
            pl.BlockSpec((IN_DIM, PROJ_DIM), lambda i: (0, 0)),
            pl.BlockSpec((1, PROJ_DIM), lambda i: (0, 0)),
        ],
        out_specs=pl.BlockSpec((128, PROJ_DIM), lambda i: (i, 0)),
        out_shape=jax.ShapeDtypeStruct((TBL_ROWS, PROJ_DIM), jnp.float32),
    )(x2d, W, b2d)


def _gather_body(table, s1, e1, qb, s2, e2, out1, out2,
                 s1v, e1v, qbv, s2v, e2v, idxs, rows,
                 gsem0, gsem1, gsem2, gsem3, wsem0, wsem1, wsem2, wsem3,
                 ssem):
    gsems = (gsem0, gsem1, gsem2, gsem3)
    wsems = (wsem0, wsem1, wsem2, wsem3)
    wid = lax.axis_index("s") * NC + lax.axis_index("c")
    qbase = wid * QPW
    cps = [
        pltpu.async_copy(s1.at[pl.ds(qbase, QPW)], s1v, ssem),
        pltpu.async_copy(e1.at[pl.ds(qbase, QPW)], e1v, ssem),
        pltpu.async_copy(qb.at[pl.ds(qbase, QPW)], qbv, ssem),
        pltpu.async_copy(s2.at[pl.ds(qbase, QPW)], s2v, ssem),
        pltpu.async_copy(e2.at[pl.ds(qbase, QPW)], e2v, ssem),
    ]
    for c in cps:
        c.wait()

    iota = lax.iota(jnp.int32, 16)
    half = lax.shift_right_logical(iota, 1)
    odd = (iota & 1) == 1

    plan = [(out1, s1v, e1v) if k < ISSUES else (out2, s2v, e2v) for k in range(TOT)]

    def compute_idx(k):
        b = k % NBUF
        _, sv, ev = plan[k]
        j = k % ISSUES
        for t in range(QPI // 16):
            qo = j * QPI + t * 16
            s = sv[pl.ds(qo, 16)]
            e = ev[pl.ds(qo, 16)]
            bb = qbv[pl.ds(qo, 16)]
            valid = e >= s
            fs = jnp.where(valid, bb * SEQ + s, ZERO_ROW)
            fe = jnp.where(valid, bb * SEQ + e, ZERO_ROW)
            lo = jnp.where(odd, _vgather(fe, half), _vgather(fs, half))
            hi = jnp.where(odd, _vgather(fe, half + 8), _vgather(fs, half + 8))
            idxs[b][pl.ds(t * 32, 16)] = lo
            idxs[b][pl.ds(t * 32 + 16, 16)] = hi

    def fire_gather(k):
        b = k % NBUF
        return pltpu.async_copy(table.at[idxs[b]], rows[b], gsems[b])

    def fire_write(k):
        b = k % NBUF
        out_ref, _, _ = plan[k]
        j = k % ISSUES
        dst = out_ref.at[pl.ds(2 * qbase + j * IDX_N, IDX_N)]
        return pltpu.async_copy(rows[b], dst, wsems[b])

    gops = [None] * TOT
    wops = [None] * TOT
    for k in range(LOOKAHEAD):
        compute_idx(k)
        gops[k] = fire_gather(k)
    for k in range(TOT):
        nk = k + LOOKAHEAD
        if nk < TOT:
            if nk >= NBUF:
                wops[nk - NBUF].wait()
            compute_idx(nk)
            gops[nk] = fire_gather(nk)
        gops[k].wait()
        wops[k] = fire_write(k)
    for k in range(TOT - NBUF, TOT):
        wops[k].wait()


def _span_gather_sc(table, s1, e1, qb, s2, e2):
    mesh = plsc.VectorSubcoreMesh(
        core_axis_name="c", subcore_axis_name="s", num_cores=NC, num_subcores=NS
    )

    def body(table, s1, e1, qb, s2, e2, out1, out2,
             s1v, e1v, qbv, s2v, e2v,
             i0, i1, i2, i3, r0, r1, r2, r3,
             gsem0, gsem1, gsem2, gsem3, wsem0, wsem1, wsem2, wsem3, ssem):
        _gather_body(table, s1, e1, qb, s2, e2, out1, out2,
                     s1v, e1v, qbv, s2v, e2v,
                     [i0, i1, i2, i3], [r0, r1, r2, r3],
                     gsem0, gsem1, gsem2, gsem3,
                     wsem0, wsem1, wsem2, wsem3, ssem)

    f = functools.partial(
        pl.kernel,
        out_type=(
            jax.ShapeDtypeStruct((2 * Q, PROJ_DIM), jnp.float32),
            jax.ShapeDtypeStruct((2 * Q, PROJ_DIM), jnp.float32),
        ),
        mesh=mesh,
        scratch_types=(
            [pltpu.VMEM((QPW,), jnp.int32)] * 5
            + [pltpu.VMEM((IDX_N,), jnp.int32)] * NBUF
            + [pltpu.VMEM((IDX_N, PROJ_DIM), jnp.float32)] * NBUF
            + [pltpu.SemaphoreType.DMA] * (2 * NBUF + 1)
        ),
    )(body)
    return f(table, s1, e1, qb, s2, e2)


def kernel(flag, encoded_input, start_ids_1, end_ids_1, query_batch_idx,
           start_ids_2, end_ids_2, W, b):
    x2d = encoded_input.reshape(ROWS, IN_DIM)
    table = _project(x2d, W, b.reshape(1, PROJ_DIM))
    s1 = start_ids_1.astype(jnp.int32)
    e1 = end_ids_1.astype(jnp.int32)
    qb = query_batch_idx.astype(jnp.int32)
    s2 = start_ids_2.astype(jnp.int32)
    e2 = end_ids_2.astype(jnp.int32)
    o1, o2 = _span_gather_sc(table, s1, e1, qb, s2, e2)
    return (o1.reshape(Q, 2 * PROJ_DIM), o2.reshape(Q, 2 * PROJ_DIM))


# speedup vs baseline: 16.5371x; 16.5371x over previous
import functools

import jax
import jax.numpy as jnp
from jax import lax
from jax.experimental import pallas as pl
from jax.experimental.pallas import tpu as pltpu
from jax.experimental.pallas import tpu_sc as plsc

BSZ, SEQ, IN_DIM, PROJ_DIM, Q = 4, 256, 768, 128, 16384
ROWS = BSZ * SEQ
TBL_ROWS = ROWS + 128
ZERO_ROW = ROWS

NC, NS = 2, 16
NW = NC * NS
QPW = Q // NW
QPI = 64
IDX_N = 2 * QPI
ISSUES = QPW // QPI
TOT = 2 * ISSUES
NBUF = 4
LOOKAHEAD = 2

_GDN = lax.GatherDimensionNumbers(
    offset_dims=(), collapsed_slice_dims=(0,), start_index_map=(0,)
)


def _vgather(x, idx):
    return lax.gather(
        x, idx[:, None], _GDN, slice_sizes=(1,),
        mode=lax.GatherScatterMode.PROMISE_IN_BOUNDS,
    )


def _proj_body(x_ref, w_ref, b_ref, o_ref):
    i = pl.program_id(0)

    @pl.when(i < ROWS // 128)
    def _():
        o_ref[...] = (
            jnp.dot(x_ref[...], w_ref[...], preferred_element_type=jnp.float32)
            + b_ref[...]
        )

    @pl.when(i >= ROWS // 128)
    def _():
        o_ref[...] = jnp.zeros_like(o_ref)


def _project(x2d, W, b2d):
    nblk = TBL_ROWS // 128
    return pl.pallas_call(
        _proj_body,
        grid=(nblk,),
        in_specs=[
            pl.BlockSpec((128, IN_DIM), lambda i: (jnp.minimum(i, ROWS // 128 - 1), 0)),
            pl.BlockSpec((IN_DIM, PROJ_DIM), lambda i: (0, 0)),
            pl.BlockSpec((1, PROJ_DIM), lambda i: (0, 0)),
        ],
        out_specs=pl.BlockSpec((128, PROJ_DIM), lambda i: (i, 0)),
        out_shape=jax.ShapeDtypeStruct((TBL_ROWS, PROJ_DIM), jnp.float32),
    )(x2d, W, b2d)


def _gather_body(table, s1, e1, qb, s2, e2, out1, out2,
                 tbl_s, s1v, e1v, qbv, s2v, e2v, idxs, rows,
                 gsem0, gsem1, gsem2, gsem3, wsem0, wsem1, wsem2, wsem3,
                 ssem):
    gsems = (gsem0, gsem1, gsem2, gsem3)
    wsems = (wsem0, wsem1, wsem2, wsem3)
    sid = lax.axis_index("s")
    wid = sid * NC + lax.axis_index("c")
    qbase = wid * QPW
    rpt = TBL_ROWS // NS
    cps = [
        pltpu.async_copy(s1.at[pl.ds(qbase, QPW)], s1v, ssem),
        pltpu.async_copy(e1.at[pl.ds(qbase, QPW)], e1v, ssem),
        pltpu.async_copy(qb.at[pl.ds(qbase, QPW)], qbv, ssem),
        pltpu.async_copy(s2.at[pl.ds(qbase, QPW)], s2v, ssem),
        pltpu.async_copy(e2.at[pl.ds(qbase, QPW)], e2v, ssem),
    ]
    pltpu.sync_copy(table.at[pl.ds(sid * rpt, rpt)], tbl_s.at[pl.ds(sid * rpt, rpt)])
    plsc.subcore_barrier()
    for c in cps:
        c.wait()

    iota = lax.iota(jnp.int32, 16)
    half = lax.shift_right_logical(iota, 1)
    odd = (iota & 1) == 1

    plan = [(out1, s1v, e1v) if k < ISSUES else (out2, s2v, e2v) for k in range(TOT)]

    def compute_idx(k):
        b = k % NBUF
        _, sv, ev = plan[k]
        j = k % ISSUES
        for t in range(QPI // 16):
            qo = j * QPI + t * 16
            s = sv[pl.ds(qo, 16)]
            e = ev[pl.ds(qo, 16)]
            bb = qbv[pl.ds(qo, 16)]
            valid = e >= s
            fs = jnp.where(valid, bb * SEQ + s, ZERO_ROW)
            fe = jnp.where(valid, bb * SEQ + e, ZERO_ROW)
            lo = jnp.where(odd, _vgather(fe, half), _vgather(fs, half))
            hi = jnp.where(odd, _vgather(fe, half + 8), _vgather(fs, half + 8))
            idxs[b][pl.ds(t * 32, 16)] = lo
            idxs[b][pl.ds(t * 32 + 16, 16)] = hi

    def fire_gather(k):
        b = k % NBUF
        return pltpu.async_copy(tbl_s.at[idxs[b]], rows[b], gsems[b])

    def fire_write(k):
        b = k % NBUF
        out_ref, _, _ = plan[k]
        j = k % ISSUES
        dst = out_ref.at[pl.ds(2 * qbase + j * IDX_N, IDX_N)]
        return pltpu.async_copy(rows[b], dst, wsems[b])

    gops = [None] * TOT
    wops = [None] * TOT
    for k in range(LOOKAHEAD):
        compute_idx(k)
        gops[k] = fire_gather(k)
    for k in range(TOT):
        nk = k + LOOKAHEAD
        if nk < TOT:
            if nk >= NBUF:
                wops[nk - NBUF].wait()
            compute_idx(nk)
            gops[nk] = fire_gather(nk)
        gops[k].wait()
        wops[k] = fire_write(k)
    for k in range(TOT - NBUF, TOT):
        wops[k].wait()


def _span_gather_sc(table, s1, e1, qb, s2, e2):
    mesh = plsc.VectorSubcoreMesh(
        core_axis_name="c", subcore_axis_name="s", num_cores=NC, num_subcores=NS
    )

    def body(table, s1, e1, qb, s2, e2, out1, out2,
             tbl_s, s1v, e1v, qbv, s2v, e2v,
             i0, i1, i2, i3, r0, r1, r2, r3,
             gsem0, gsem1, gsem2, gsem3, wsem0, wsem1, wsem2, wsem3, ssem):
        _gather_body(table, s1, e1, qb, s2, e2, out1, out2,
                     tbl_s, s1v, e1v, qbv, s2v, e2v,
                     [i0, i1, i2, i3], [r0, r1, r2, r3],
                     gsem0, gsem1, gsem2, gsem3,
                     wsem0, wsem1, wsem2, wsem3, ssem)

    f = functools.partial(
        pl.kernel,
        out_type=(
            jax.ShapeDtypeStruct((2 * Q, PROJ_DIM), jnp.float32),
            jax.ShapeDtypeStruct((2 * Q, PROJ_DIM), jnp.float32),
        ),
        mesh=mesh,
        scratch_types=(
            [pltpu.VMEM_SHARED((TBL_ROWS, PROJ_DIM), jnp.float32)]
            + [pltpu.VMEM((QPW,), jnp.int32)] * 5
            + [pltpu.VMEM((IDX_N,), jnp.int32)] * NBUF
            + [pltpu.VMEM((IDX_N, PROJ_DIM), jnp.float32)] * NBUF
            + [pltpu.SemaphoreType.DMA] * (2 * NBUF + 1)
        ),
    )(body)
    return f(table, s1, e1, qb, s2, e2)


def kernel(flag, encoded_input, start_ids_1, end_ids_1, query_batch_idx,
           start_ids_2, end_ids_2, W, b):
    x2d = encoded_input.reshape(ROWS, IN_DIM)
    table = _project(x2d, W, b.reshape(1, PROJ_DIM))
    s1 = start_ids_1.astype(jnp.int32)
    e1 = end_ids_1.astype(jnp.int32)
    qb = query_batch_idx.astype(jnp.int32)
    s2 = start_ids_2.astype(jnp.int32)
    e2 = end_ids_2.astype(jnp.int32)
    o1, o2 = _span_gather_sc(table, s1, e1, qb, s2, e2)
    return (o1.reshape(Q, 2 * PROJ_DIM), o2.reshape(Q, 2 * PROJ_DIM))


# speedup vs baseline: 16.5919x; 1.0033x over previous
import functools

import jax
import jax.numpy as jnp
from jax import lax
from jax.experimental import pallas as pl
from jax.experimental.pallas import tpu as pltpu
from jax.experimental.pallas import tpu_sc as plsc

BSZ, SEQ, IN_DIM, PROJ_DIM, Q = 4, 256, 768, 128, 16384
ROWS = BSZ * SEQ
TBL_ROWS = ROWS + 128
ZERO_ROW = ROWS

NC, NS = 2, 16
NW = NC * NS
QPW = Q // NW
QPI = 64
IDX_N = 2 * QPI
ISSUES = QPW // QPI
TOT = 2 * ISSUES
NBUF = 6
LOOKAHEAD = 3

_GDN = lax.GatherDimensionNumbers(
    offset_dims=(), collapsed_slice_dims=(0,), start_index_map=(0,)
)


def _vgather(x, idx):
    return lax.gather(
        x, idx[:, None], _GDN, slice_sizes=(1,),
        mode=lax.GatherScatterMode.PROMISE_IN_BOUNDS,
    )


def _proj_body(x_ref, w_ref, b_ref, o_ref):
    i = pl.program_id(0)

    @pl.when(i < ROWS // 128)
    def _():
        o_ref[...] = (
            jnp.dot(x_ref[...], w_ref[...], preferred_element_type=jnp.float32)
            + b_ref[...]
        )

    @pl.when(i >= ROWS // 128)
    def _():
        o_ref[...] = jnp.zeros_like(o_ref)


def _project(x2d, W, b2d):
    nblk = TBL_ROWS // 128
    return pl.pallas_call(
        _proj_body,
        grid=(nblk,),
        in_specs=[
            pl.BlockSpec((128, IN_DIM), lambda i: (jnp.minimum(i, ROWS // 128 - 1), 0)),
            pl.BlockSpec((IN_DIM, PROJ_DIM), lambda i: (0, 0)),
            pl.BlockSpec((1, PROJ_DIM), lambda i: (0, 0)),
        ],
        out_specs=pl.BlockSpec((128, PROJ_DIM), lambda i: (i, 0)),
        out_shape=jax.ShapeDtypeStruct((TBL_ROWS, PROJ_DIM), jnp.float32),
    )(x2d, W, b2d)


def _gather_body(table, s1, e1, qb, s2, e2, out1, out2,
                 tbl_s, s1v, e1v, qbv, s2v, e2v, idxs, rows,
                 gsems, wsems, ssem):
    sid = lax.axis_index("s")
    wid = sid * NC + lax.axis_index("c")
    qbase = wid * QPW
    rpt = TBL_ROWS // NS
    cps = [
        pltpu.async_copy(s1.at[pl.ds(qbase, QPW)], s1v, ssem),
        pltpu.async_copy(e1.at[pl.ds(qbase, QPW)], e1v, ssem),
        pltpu.async_copy(qb.at[pl.ds(qbase, QPW)], qbv, ssem),
        pltpu.async_copy(s2.at[pl.ds(qbase, QPW)], s2v, ssem),
        pltpu.async_copy(e2.at[pl.ds(qbase, QPW)], e2v, ssem),
    ]
    pltpu.sync_copy(table.at[pl.ds(sid * rpt, rpt)], tbl_s.at[pl.ds(sid * rpt, rpt)])
    plsc.subcore_barrier()
    for c in cps:
        c.wait()

    iota = lax.iota(jnp.int32, 16)
    half = lax.shift_right_logical(iota, 1)
    odd = (iota & 1) == 1

    plan = [(out1, s1v, e1v) if k < ISSUES else (out2, s2v, e2v) for k in range(TOT)]

    def compute_idx(k):
        b = k % NBUF
        _, sv, ev = plan[k]
        j = k % ISSUES
        for t in range(QPI // 16):
            qo = j * QPI + t * 16
            s = sv[pl.ds(qo, 16)]
            e = ev[pl.ds(qo, 16)]
            bb = qbv[pl.ds(qo, 16)]
            valid = e >= s
            fs = jnp.where(valid, bb * SEQ + s, ZERO_ROW)
            fe = jnp.where(valid, bb * SEQ + e, ZERO_ROW)
            lo = jnp.where(odd, _vgather(fe, half), _vgather(fs, half))
            hi = jnp.where(odd, _vgather(fe, half + 8), _vgather(fs, half + 8))
            idxs[b][pl.ds(t * 32, 16)] = lo
            idxs[b][pl.ds(t * 32 + 16, 16)] = hi

    def fire_gather(k):
        b = k % NBUF
        return pltpu.async_copy(tbl_s.at[idxs[b]], rows[b], gsems[b])

    def fire_write(k):
        b = k % NBUF
        out_ref, _, _ = plan[k]
        j = k % ISSUES
        dst = out_ref.at[pl.ds(2 * qbase + j * IDX_N, IDX_N)]
        return pltpu.async_copy(rows[b], dst, wsems[b])

    gops = [None] * TOT
    wops = [None] * TOT
    for k in range(LOOKAHEAD):
        compute_idx(k)
        gops[k] = fire_gather(k)
    for k in range(TOT):
        nk = k + LOOKAHEAD
        if nk < TOT:
            if nk >= NBUF:
                wops[nk - NBUF].wait()
            compute_idx(nk)
            gops[nk] = fire_gather(nk)
        gops[k].wait()
        wops[k] = fire_write(k)
    for k in range(TOT - NBUF, TOT):
        wops[k].wait()


def _span_gather_sc(table, s1, e1, qb, s2, e2):
    mesh = plsc.VectorSubcoreMesh(
        core_axis_name="c", subcore_axis_name="s", num_cores=NC, num_subcores=NS
    )

    def body(table, s1, e1, qb, s2, e2, out1, out2, tbl_s,
             s1v, e1v, qbv, s2v, e2v, *rest):
        idxs = list(rest[:NBUF])
        rows = list(rest[NBUF:2 * NBUF])
        gsems = list(rest[2 * NBUF:3 * NBUF])
        wsems = list(rest[3 * NBUF:4 * NBUF])
        ssem = rest[4 * NBUF]
        _gather_body(table, s1, e1, qb, s2, e2, out1, out2,
                     tbl_s, s1v, e1v, qbv, s2v, e2v,
                     idxs, rows, gsems, wsems, ssem)

    f = functools.partial(
        pl.kernel,
        out_type=(
            jax.ShapeDtypeStruct((2 * Q, PROJ_DIM), jnp.float32),
            jax.ShapeDtypeStruct((2 * Q, PROJ_DIM), jnp.float32),
        ),
        mesh=mesh,
        scratch_types=(
            [pltpu.VMEM_SHARED((TBL_ROWS, PROJ_DIM), jnp.float32)]
            + [pltpu.VMEM((QPW,), jnp.int32)] * 5
            + [pltpu.VMEM((IDX_N,), jnp.int32)] * NBUF
            + [pltpu.VMEM((IDX_N, PROJ_DIM), jnp.float32)] * NBUF
            + [pltpu.SemaphoreType.DMA] * (2 * NBUF + 1)
        ),
    )(body)
    return f(table, s1, e1, qb, s2, e2)


def kernel(flag, encoded_input, start_ids_1, end_ids_1, query_batch_idx,
           start_ids_2, end_ids_2, W, b):
    x2d = encoded_input.reshape(ROWS, IN_DIM)
    table = _project(x2d, W, b.reshape(1, PROJ_DIM))
    s1 = start_ids_1.astype(jnp.int32)
    e1 = end_ids_1.astype(jnp.int32)
    qb = query_batch_idx.astype(jnp.int32)
    s2 = start_ids_2.astype(jnp.int32)
    e2 = end_ids_2.astype(jnp.int32)
    o1, o2 = _span_gather_sc(table, s1, e1, qb, s2, e2)
    return (o1.reshape(Q, 2 * PROJ_DIM), o2.reshape(Q, 2 * PROJ_DIM))


# speedup vs baseline: 31.0605x; 1.8720x over previous
import functools

import jax
import jax.numpy as jnp
from jax import lax
from jax.experimental import pallas as pl
from jax.experimental.pallas import tpu as pltpu
from jax.experimental.pallas import tpu_sc as plsc

BSZ, SEQ, IN_DIM, PROJ_DIM, Q = 4, 256, 768, 128, 16384
ROWS = BSZ * SEQ
TBL_ROWS = ROWS + 128
ZERO_ROW = ROWS

NC, NS = 2, 16
NW = NC * NS
QPW = Q // NW
QPI = 64
IDX_N = 2 * QPI
ISSUES = QPW // QPI
TOT = 2 * ISSUES
NBUF = 6
LOOKAHEAD = 3

_GDN = lax.GatherDimensionNumbers(
    offset_dims=(), collapsed_slice_dims=(0,), start_index_map=(0,)
)


def _vgather(x, idx):
    return lax.gather(
        x, idx[:, None], _GDN, slice_sizes=(1,),
        mode=lax.GatherScatterMode.PROMISE_IN_BOUNDS,
    )


def _proj_body(x_ref, w_ref, b_ref, o_ref):
    i = pl.program_id(0)

    @pl.when(i < ROWS // 128)
    def _():
        o_ref[...] = (
            jnp.dot(x_ref[...], w_ref[...], preferred_element_type=jnp.float32)
            + b_ref[...]
        )

    @pl.when(i >= ROWS // 128)
    def _():
        o_ref[...] = jnp.zeros_like(o_ref)


def _project(x2d, W, b2d):
    nblk = TBL_ROWS // 128
    return pl.pallas_call(
        _proj_body,
        grid=(nblk,),
        in_specs=[
            pl.BlockSpec((128, IN_DIM), lambda i: (jnp.minimum(i, ROWS // 128 - 1), 0)),
            pl.BlockSpec((IN_DIM, PROJ_DIM), lambda i: (0, 0)),
            pl.BlockSpec((1, PROJ_DIM), lambda i: (0, 0)),
        ],
        out_specs=pl.BlockSpec((128, PROJ_DIM), lambda i: (i, 0)),
        out_shape=jax.ShapeDtypeStruct((TBL_ROWS, PROJ_DIM), jnp.float32),
    )(x2d, W, b2d)


def _gather_body(table, s1, e1, qb, s2, e2, out1, out2,
                 tbl_s, s1v, e1v, qbv, s2v, e2v, idxs, rows,
                 gsems, wsems, ssem):
    sid = lax.axis_index("s")
    wid = sid * NC + lax.axis_index("c")
    qbase = wid * QPW
    rpt = TBL_ROWS // NS
    cps = [
        pltpu.async_copy(s1.at[pl.ds(qbase, QPW)], s1v, ssem),
        pltpu.async_copy(e1.at[pl.ds(qbase, QPW)], e1v, ssem),
        pltpu.async_copy(qb.at[pl.ds(qbase, QPW)], qbv, ssem),
        pltpu.async_copy(s2.at[pl.ds(qbase, QPW)], s2v, ssem),
        pltpu.async_copy(e2.at[pl.ds(qbase, QPW)], e2v, ssem),
    ]
    pltpu.sync_copy(table.at[pl.ds(sid * rpt, rpt)], tbl_s.at[pl.ds(sid * rpt, rpt)])
    plsc.subcore_barrier()
    for c in cps:
        c.wait()

    plan = [(out1, s1v, e1v) if k < ISSUES else (out2, s2v, e2v) for k in range(TOT)]

    def compute_idx(k):
        b = k % NBUF
        _, sv, ev = plan[k]
        j = k % ISSUES
        for t in range(QPI // 16):
            qo = j * QPI + t * 16
            s = sv[pl.ds(qo, 16)]
            e = ev[pl.ds(qo, 16)]
            bb = qbv[pl.ds(qo, 16)]
            valid = e >= s
            fs = jnp.where(valid, bb * SEQ + s, ZERO_ROW)
            fe = jnp.where(valid, bb * SEQ + e, ZERO_ROW)
            idxs[b][pl.ds(t * 16, 16)] = fs
            idxs[b][pl.ds(QPI + t * 16, 16)] = fe

    def fire_gather(k):
        b = k % NBUF
        return pltpu.async_copy(tbl_s.at[idxs[b]], rows[b], gsems[b])

    def fire_write(k):
        b = k % NBUF
        out_ref, _, _ = plan[k]
        j = k % ISSUES
        qrow = qbase + j * QPI
        c1 = pltpu.async_copy(
            rows[b].at[pl.ds(0, QPI)],
            out_ref.at[pl.ds(qrow, QPI), pl.ds(0, PROJ_DIM)], wsems[b])
        c2 = pltpu.async_copy(
            rows[b].at[pl.ds(QPI, QPI)],
            out_ref.at[pl.ds(qrow, QPI), pl.ds(PROJ_DIM, PROJ_DIM)], wsems[b])
        return (c1, c2)

    gops = [None] * TOT
    wops = [None] * TOT
    for k in range(LOOKAHEAD):
        compute_idx(k)
        gops[k] = fire_gather(k)
    for k in range(TOT):
        nk = k + LOOKAHEAD
        if nk < TOT:
            if nk >= NBUF:
                for c in wops[nk - NBUF]:
                    c.wait()
            compute_idx(nk)
            gops[nk] = fire_gather(nk)
        gops[k].wait()
        wops[k] = fire_write(k)
    for k in range(TOT - NBUF, TOT):
        for c in wops[k]:
            c.wait()


def _span_gather_sc(table, s1, e1, qb, s2, e2):
    mesh = plsc.VectorSubcoreMesh(
        core_axis_name="c", subcore_axis_name="s", num_cores=NC, num_subcores=NS
    )

    def body(table, s1, e1, qb, s2, e2, out1, out2, tbl_s,
             s1v, e1v, qbv, s2v, e2v, *rest):
        idxs = list(rest[:NBUF])
        rows = list(rest[NBUF:2 * NBUF])
        gsems = list(rest[2 * NBUF:3 * NBUF])
        wsems = list(rest[3 * NBUF:4 * NBUF])
        ssem = rest[4 * NBUF]
        _gather_body(table, s1, e1, qb, s2, e2, out1, out2,
                     tbl_s, s1v, e1v, qbv, s2v, e2v,
                     idxs, rows, gsems, wsems, ssem)

    f = functools.partial(
        pl.kernel,
        out_type=(
            jax.ShapeDtypeStruct((Q, 2 * PROJ_DIM), jnp.float32),
            jax.ShapeDtypeStruct((Q, 2 * PROJ_DIM), jnp.float32),
        ),
        mesh=mesh,
        scratch_types=(
            [pltpu.VMEM_SHARED((TBL_ROWS, PROJ_DIM), jnp.float32)]
            + [pltpu.VMEM((QPW,), jnp.int32)] * 5
            + [pltpu.VMEM((IDX_N,), jnp.int32)] * NBUF
            + [pltpu.VMEM((IDX_N, PROJ_DIM), jnp.float32)] * NBUF
            + [pltpu.SemaphoreType.DMA] * (2 * NBUF + 1)
        ),
    )(body)
    return f(table, s1, e1, qb, s2, e2)


def kernel(flag, encoded_input, start_ids_1, end_ids_1, query_batch_idx,
           start_ids_2, end_ids_2, W, b):
    x2d = encoded_input.reshape(ROWS, IN_DIM)
    table = _project(x2d, W, b.reshape(1, PROJ_DIM))
    s1 = start_ids_1.astype(jnp.int32)
    e1 = end_ids_1.astype(jnp.int32)
    qb = query_batch_idx.astype(jnp.int32)
    s2 = start_ids_2.astype(jnp.int32)
    e2 = end_ids_2.astype(jnp.int32)
    return _span_gather_sc(table, s1, e1, qb, s2, e2)


# speedup vs baseline: 32.9663x; 1.0614x over previous
import functools

import jax
import jax.numpy as jnp
from jax import lax
from jax.experimental import pallas as pl
from jax.experimental.pallas import tpu as pltpu
from jax.experimental.pallas import tpu_sc as plsc

BSZ, SEQ, IN_DIM, PROJ_DIM, Q = 4, 256, 768, 128, 16384
ROWS = BSZ * SEQ
TBL_ROWS = ROWS + 8
ZERO_ROW = ROWS

NC, NS = 2, 16
NW = NC * NS
QPW = Q // NW
QPI = 64
IDX_N = 2 * QPI
ISSUES = QPW // QPI
TOT = 2 * ISSUES
NBUF = 6
LOOKAHEAD = 3

_GDN = lax.GatherDimensionNumbers(
    offset_dims=(), collapsed_slice_dims=(0,), start_index_map=(0,)
)


def _vgather(x, idx):
    return lax.gather(
        x, idx[:, None], _GDN, slice_sizes=(1,),
        mode=lax.GatherScatterMode.PROMISE_IN_BOUNDS,
    )


PROJ_BLK = 512


def _proj_body(x_ref, w_ref, b_ref, o_ref):
    o_ref[...] = (
        jnp.dot(x_ref[...], w_ref[...], preferred_element_type=jnp.float32)
        + b_ref[...]
    )


def _project(x2d, W, b2d):
    return pl.pallas_call(
        _proj_body,
        grid=(ROWS // PROJ_BLK,),
        in_specs=[
            pl.BlockSpec((PROJ_BLK, IN_DIM), lambda i: (i, 0)),
            pl.BlockSpec((IN_DIM, PROJ_DIM), lambda i: (0, 0)),
            pl.BlockSpec((1, PROJ_DIM), lambda i: (0, 0)),
        ],
        out_specs=pl.BlockSpec((PROJ_BLK, PROJ_DIM), lambda i: (i, 0)),
        out_shape=jax.ShapeDtypeStruct((ROWS, PROJ_DIM), jnp.float32),
    )(x2d, W, b2d)


def _gather_body(table, s1, e1, qb, s2, e2, out1, out2,
                 tbl_s, zrow, s1v, e1v, qbv, s2v, e2v, idxs, rows,
                 gsems, wsems, ssem):
    sid = lax.axis_index("s")
    wid = sid * NC + lax.axis_index("c")
    qbase = wid * QPW
    rpt = ROWS // NS
    cps = [
        pltpu.async_copy(s1.at[pl.ds(qbase, QPW)], s1v, ssem),
        pltpu.async_copy(e1.at[pl.ds(qbase, QPW)], e1v, ssem),
        pltpu.async_copy(qb.at[pl.ds(qbase, QPW)], qbv, ssem),
        pltpu.async_copy(s2.at[pl.ds(qbase, QPW)], s2v, ssem),
        pltpu.async_copy(e2.at[pl.ds(qbase, QPW)], e2v, ssem),
    ]
    pltpu.sync_copy(table.at[pl.ds(sid * rpt, rpt)], tbl_s.at[pl.ds(sid * rpt, rpt)])
    for t in range(PROJ_DIM // 16):
        zrow[pl.ds(t * 16, 16)] = jnp.zeros((16,), jnp.float32)
    pltpu.sync_copy(zrow, tbl_s.at[ZERO_ROW])
    plsc.subcore_barrier()
    for c in cps:
        c.wait()

    plan = [(out1, s1v, e1v) if k < ISSUES else (out2, s2v, e2v) for k in range(TOT)]

    def compute_idx(k):
        b = k % NBUF
        _, sv, ev = plan[k]
        j = k % ISSUES
        for t in range(QPI // 16):
            qo = j * QPI + t * 16
            s = sv[pl.ds(qo, 16)]
            e = ev[pl.ds(qo, 16)]
            bb = qbv[pl.ds(qo, 16)]
            valid = e >= s
            fs = jnp.where(valid, bb * SEQ + s, ZERO_ROW)
            fe = jnp.where(valid, bb * SEQ + e, ZERO_ROW)
            idxs[b][pl.ds(t * 16, 16)] = fs
            idxs[b][pl.ds(QPI + t * 16, 16)] = fe

    def fire_gather(k):
        b = k % NBUF
        return pltpu.async_copy(tbl_s.at[idxs[b]], rows[b], gsems[b])

    def fire_write(k):
        b = k % NBUF
        out_ref, _, _ = plan[k]
        j = k % ISSUES
        qrow = qbase + j * QPI
        c1 = pltpu.async_copy(
            rows[b].at[pl.ds(0, QPI)],
            out_ref.at[pl.ds(qrow, QPI), pl.ds(0, PROJ_DIM)], wsems[b])
        c2 = pltpu.async_copy(
            rows[b].at[pl.ds(QPI, QPI)],
            out_ref.at[pl.ds(qrow, QPI), pl.ds(PROJ_DIM, PROJ_DIM)], wsems[b])
        return (c1, c2)

    gops = [None] * TOT
    wops = [None] * TOT
    for k in range(LOOKAHEAD):
        compute_idx(k)
        gops[k] = fire_gather(k)
    for k in range(TOT):
        nk = k + LOOKAHEAD
        if nk < TOT:
            if nk >= NBUF:
                for c in wops[nk - NBUF]:
                    c.wait()
            compute_idx(nk)
            gops[nk] = fire_gather(nk)
        gops[k].wait()
        wops[k] = fire_write(k)
    for k in range(TOT - NBUF, TOT):
        for c in wops[k]:
            c.wait()


def _span_gather_sc(table, s1, e1, qb, s2, e2):
    mesh = plsc.VectorSubcoreMesh(
        core_axis_name="c", subcore_axis_name="s", num_cores=NC, num_subcores=NS
    )

    def body(table, s1, e1, qb, s2, e2, out1, out2, tbl_s, zrow,
             s1v, e1v, qbv, s2v, e2v, *rest):
        idxs = list(rest[:NBUF])
        rows = list(rest[NBUF:2 * NBUF])
        gsems = list(rest[2 * NBUF:3 * NBUF])
        wsems = list(rest[3 * NBUF:4 * NBUF])
        ssem = rest[4 * NBUF]
        _gather_body(table, s1, e1, qb, s2, e2, out1, out2,
                     tbl_s, zrow, s1v, e1v, qbv, s2v, e2v,
                     idxs, rows, gsems, wsems, ssem)

    f = functools.partial(
        pl.kernel,
        out_type=(
            jax.ShapeDtypeStruct((Q, 2 * PROJ_DIM), jnp.float32),
            jax.ShapeDtypeStruct((Q, 2 * PROJ_DIM), jnp.float32),
        ),
        mesh=mesh,
        scratch_types=(
            [pltpu.VMEM_SHARED((TBL_ROWS, PROJ_DIM), jnp.float32)]
            + [pltpu.VMEM((PROJ_DIM,), jnp.float32)]
            + [pltpu.VMEM((QPW,), jnp.int32)] * 5
            + [pltpu.VMEM((IDX_N,), jnp.int32)] * NBUF
            + [pltpu.VMEM((IDX_N, PROJ_DIM), jnp.float32)] * NBUF
            + [pltpu.SemaphoreType.DMA] * (2 * NBUF + 1)
        ),
    )(body)
    return f(table, s1, e1, qb, s2, e2)


def kernel(flag, encoded_input, start_ids_1, end_ids_1, query_batch_idx,
           start_ids_2, end_ids_2, W, b):
    x2d = encoded_input.reshape(ROWS, IN_DIM)
    table = _project(x2d, W, b.reshape(1, PROJ_DIM))
    s1 = start_ids_1.astype(jnp.int32)
    e1 = end_ids_1.astype(jnp.int32)
    qb = query_batch_idx.astype(jnp.int32)
    s2 = start_ids_2.astype(jnp.int32)
    e2 = end_ids_2.astype(jnp.int32)
    return _span_gather_sc(table, s1, e1, qb, s2, e2)


# speedup vs baseline: 33.0063x; 1.0012x over previous
import functools

import jax
import jax.numpy as jnp
from jax import lax
from jax.experimental import pallas as pl
from jax.experimental.pallas import tpu as pltpu
from jax.experimental.pallas import tpu_sc as plsc

BSZ, SEQ, IN_DIM, PROJ_DIM, Q = 4, 256, 768, 128, 16384
ROWS = BSZ * SEQ
TBL_ROWS = ROWS + 8
ZERO_ROW = ROWS

NC, NS = 2, 16
NW = NC * NS
QPW = Q // NW
QPI = 64
IDX_N = 2 * QPI
ISSUES = QPW // QPI
TOT = 2 * ISSUES
NBUF = 6
LOOKAHEAD = 3

_GDN = lax.GatherDimensionNumbers(
    offset_dims=(), collapsed_slice_dims=(0,), start_index_map=(0,)
)


def _vgather(x, idx):
    return lax.gather(
        x, idx[:, None], _GDN, slice_sizes=(1,),
        mode=lax.GatherScatterMode.PROMISE_IN_BOUNDS,
    )


PROJ_BLK = 512


def _proj_body(x_ref, w_ref, b_ref, o_ref):
    o_ref[...] = (
        jnp.dot(x_ref[...].astype(jnp.bfloat16), w_ref[...].astype(jnp.bfloat16),
                preferred_element_type=jnp.float32)
        + b_ref[...]
    )


def _project(x2d, W, b2d):
    return pl.pallas_call(
        _proj_body,
        grid=(ROWS // PROJ_BLK,),
        in_specs=[
            pl.BlockSpec((PROJ_BLK, IN_DIM), lambda i: (i, 0)),
            pl.BlockSpec((IN_DIM, PROJ_DIM), lambda i: (0, 0)),
            pl.BlockSpec((1, PROJ_DIM), lambda i: (0, 0)),
        ],
        out_specs=pl.BlockSpec((PROJ_BLK, PROJ_DIM), lambda i: (i, 0)),
        out_shape=jax.ShapeDtypeStruct((ROWS, PROJ_DIM), jnp.float32),
    )(x2d, W, b2d)


def _gather_body(table, s1, e1, qb, s2, e2, out1, out2,
                 tbl_s, zrow, s1v, e1v, qbv, s2v, e2v, idxs, rows,
                 gsems, wsems, ssem):
    sid = lax.axis_index("s")
    wid = sid * NC + lax.axis_index("c")
    qbase = wid * QPW
    rpt = ROWS // NS
    cps = [
        pltpu.async_copy(s1.at[pl.ds(qbase, QPW)], s1v, ssem),
        pltpu.async_copy(e1.at[pl.ds(qbase, QPW)], e1v, ssem),
        pltpu.async_copy(qb.at[pl.ds(qbase, QPW)], qbv, ssem),
        pltpu.async_copy(s2.at[pl.ds(qbase, QPW)], s2v, ssem),
        pltpu.async_copy(e2.at[pl.ds(qbase, QPW)], e2v, ssem),
    ]
    pltpu.sync_copy(table.at[pl.ds(sid * rpt, rpt)], tbl_s.at[pl.ds(sid * rpt, rpt)])
    for t in range(PROJ_DIM // 16):
        zrow[pl.ds(t * 16, 16)] = jnp.zeros((16,), jnp.float32)
    pltpu.sync_copy(zrow, tbl_s.at[ZERO_ROW])
    plsc.subcore_barrier()
    for c in cps:
        c.wait()

    plan = [(out1, s1v, e1v) if k < ISSUES else (out2, s2v, e2v) for k in range(TOT)]

    def compute_idx(k):
        b = k % NBUF
        _, sv, ev = plan[k]
        j = k % ISSUES
        for t in range(QPI // 16):
            qo = j * QPI + t * 16
            s = sv[pl.ds(qo, 16)]
            e = ev[pl.ds(qo, 16)]
            bb = qbv[pl.ds(qo, 16)]
            valid = e >= s
            fs = jnp.where(valid, bb * SEQ + s, ZERO_ROW)
            fe = jnp.where(valid, bb * SEQ + e, ZERO_ROW)
            idxs[b][pl.ds(t * 16, 16)] = fs
            idxs[b][pl.ds(QPI + t * 16, 16)] = fe

    def fire_gather(k):
        b = k % NBUF
        return pltpu.async_copy(tbl_s.at[idxs[b]], rows[b], gsems[b])

    def fire_write(k):
        b = k % NBUF
        out_ref, _, _ = plan[k]
        j = k % ISSUES
        qrow = qbase + j * QPI
        c1 = pltpu.async_copy(
            rows[b].at[pl.ds(0, QPI)],
            out_ref.at[pl.ds(qrow, QPI), pl.ds(0, PROJ_DIM)], wsems[b])
        c2 = pltpu.async_copy(
            rows[b].at[pl.ds(QPI, QPI)],
            out_ref.at[pl.ds(qrow, QPI), pl.ds(PROJ_DIM, PROJ_DIM)], wsems[b])
        return (c1, c2)

    gops = [None] * TOT
    wops = [None] * TOT
    for k in range(LOOKAHEAD):
        compute_idx(k)
        gops[k] = fire_gather(k)
    for k in range(TOT):
        nk = k + LOOKAHEAD
        if nk < TOT:
            if nk >= NBUF:
                for c in wops[nk - NBUF]:
                    c.wait()
            compute_idx(nk)
            gops[nk] = fire_gather(nk)
        gops[k].wait()
        wops[k] = fire_write(k)
    for k in range(TOT - NBUF, TOT):
        for c in wops[k]:
            c.wait()


def _span_gather_sc(table, s1, e1, qb, s2, e2):
    mesh = plsc.VectorSubcoreMesh(
        core_axis_name="c", subcore_axis_name="s", num_cores=NC, num_subcores=NS
    )

    def body(table, s1, e1, qb, s2, e2, out1, out2, tbl_s, zrow,
             s1v, e1v, qbv, s2v, e2v, *rest):
        idxs = list(rest[:NBUF])
        rows = list(rest[NBUF:2 * NBUF])
        gsems = list(rest[2 * NBUF:3 * NBUF])
        wsems = list(rest[3 * NBUF:4 * NBUF])
        ssem = rest[4 * NBUF]
        _gather_body(table, s1, e1, qb, s2, e2, out1, out2,
                     tbl_s, zrow, s1v, e1v, qbv, s2v, e2v,
                     idxs, rows, gsems, wsems, ssem)

    f = functools.partial(
        pl.kernel,
        out_type=(
            jax.ShapeDtypeStruct((Q, 2 * PROJ_DIM), jnp.float32),
            jax.ShapeDtypeStruct((Q, 2 * PROJ_DIM), jnp.float32),
        ),
        mesh=mesh,
        scratch_types=(
            [pltpu.VMEM_SHARED((TBL_ROWS, PROJ_DIM), jnp.float32)]
            + [pltpu.VMEM((PROJ_DIM,), jnp.float32)]
            + [pltpu.VMEM((QPW,), jnp.int32)] * 5
            + [pltpu.VMEM((IDX_N,), jnp.int32)] * NBUF
            + [pltpu.VMEM((IDX_N, PROJ_DIM), jnp.float32)] * NBUF
            + [pltpu.SemaphoreType.DMA] * (2 * NBUF + 1)
        ),
    )(body)
    return f(table, s1, e1, qb, s2, e2)


def kernel(flag, encoded_input, start_ids_1, end_ids_1, query_batch_idx,
           start_ids_2, end_ids_2, W, b):
    x2d = encoded_input.reshape(ROWS, IN_DIM)
    table = _project(x2d, W, b.reshape(1, PROJ_DIM))
    s1 = start_ids_1.astype(jnp.int32)
    e1 = end_ids_1.astype(jnp.int32)
    qb = query_batch_idx.astype(jnp.int32)
    s2 = start_ids_2.astype(jnp.int32)
    e2 = end_ids_2.astype(jnp.int32)
    return _span_gather_sc(table, s1, e1, qb, s2, e2)
